# Initial kernel scaffold; baseline (speedup 1.0000x reference)
import jax
import jax.numpy as jnp
from jax.experimental import pallas as pl


def kernel(x, edge_index, W1, b1, W2, b2):
    raise NotImplementedError("write your pallas kernel here")


# speedup vs baseline: 237.8415x; 237.8415x over previous
import functools

import jax
import jax.numpy as jnp
from jax import lax
from jax.experimental import pallas as pl
from jax.experimental.pallas import tpu as pltpu
from jax.experimental.pallas import tpu_sc as plsc

N_NODES = 100000
N_PAD = 100352
ROWS2D = 784
WSEG = N_PAD // 16

LANES = 16
NC = 2
NS = 16
NW = NC * NS
CH_ROWS = 125
CHUNK = CH_ROWS * LANES

_f32 = jnp.float32


def _mesh():
    return plsc.VectorSubcoreMesh(core_axis_name="c", subcore_axis_name="s")


def _zero_fill(wbuf):
    z = jnp.zeros((LANES,), _f32)
    for k in range(WSEG // LANES):
        wbuf[pl.ds(k * LANES, LANES)] = z


def _acc_init(s, wbuf, acc):
    _zero_fill(wbuf)
    pltpu.sync_copy(wbuf, acc.at[pl.ds(s * WSEG, WSEG)])


def _acc_writeout(c, s, wbuf, acc, out_hbm):
    pltpu.sync_copy(acc.at[pl.ds(s * WSEG, WSEG)], wbuf)
    pltpu.sync_copy(wbuf, out_hbm.at[c, pl.ds(s * WSEG, WSEG)])


def _deg_body(nchunks, dst_hbm, out_hbm, dbuf, vbuf, wbuf, acc):
    c = lax.axis_index("c")
    s = lax.axis_index("s")
    one = jnp.ones((LANES,), _f32)
    for k in range(CH_ROWS):
        vbuf[pl.ds(k * LANES, LANES)] = one
    _acc_init(s, wbuf, acc)
    plsc.subcore_barrier()
    base0 = (c * NS + s) * (nchunks * CHUNK)

    def chunk(i, carry):
        r = base0 + i * CHUNK
        pltpu.sync_copy(dst_hbm.at[pl.ds(r, CHUNK)], dbuf)
        pltpu.sync_copy(vbuf, acc.at[dbuf], add=True)
        return carry

    lax.fori_loop(0, nchunks, chunk, 0)
    plsc.subcore_barrier()
    _acc_writeout(c, s, wbuf, acc, out_hbm)


def _edge_body(nchunks, vals_hbm, src_hbm, dst_hbm, out_hbm,
               sbuf, dbuf, vbuf, wbuf, s_local, acc):
    c = lax.axis_index("c")
    s = lax.axis_index("s")
    pltpu.sync_copy(vals_hbm, s_local)
    _acc_init(s, wbuf, acc)
    plsc.subcore_barrier()
    base0 = (c * NS + s) * (nchunks * CHUNK)

    def chunk(i, carry):
        r = base0 + i * CHUNK
        pltpu.sync_copy(src_hbm.at[pl.ds(r, CHUNK)], sbuf)
        pltpu.sync_copy(dst_hbm.at[pl.ds(r, CHUNK)], dbuf)
        for k in range(CH_ROWS):
            idx = sbuf[pl.ds(k * LANES, LANES)]
            vbuf[pl.ds(k * LANES, LANES)] = plsc.load_gather(s_local, [idx])
        pltpu.sync_copy(vbuf, acc.at[dbuf], add=True)
        return carry

    lax.fori_loop(0, nchunks, chunk, 0)
    plsc.subcore_barrier()
    _acc_writeout(c, s, wbuf, acc, out_hbm)


@functools.partial(jax.jit, static_argnames=("nchunks",))
def _deg_pass(dst2d, nchunks):
    return pl.kernel(
        functools.partial(_deg_body, nchunks),
        out_type=jax.ShapeDtypeStruct((NC, N_PAD), _f32),
        mesh=_mesh(),
        scratch_types=[
            pltpu.VMEM((CHUNK,), jnp.int32),
            pltpu.VMEM((CHUNK,), _f32),
            pltpu.VMEM((WSEG,), _f32),
            pltpu.VMEM_SHARED((N_PAD,), _f32),
        ],
    )(dst2d)


@functools.partial(jax.jit, static_argnames=("nchunks",))
def _edge_pass(vals, src2d, dst2d, nchunks):
    return pl.kernel(
        functools.partial(_edge_body, nchunks),
        out_type=jax.ShapeDtypeStruct((NC, N_PAD), _f32),
        mesh=_mesh(),
        compiler_params=pltpu.CompilerParams(needs_layout_passes=False),
        scratch_types=[
            pltpu.VMEM((CHUNK,), jnp.int32),
            pltpu.VMEM((CHUNK,), jnp.int32),
            pltpu.VMEM((CHUNK,), _f32),
            pltpu.VMEM((WSEG,), _f32),
            pltpu.VMEM((N_PAD,), _f32),
            pltpu.VMEM_SHARED((N_PAD,), _f32),
        ],
    )(vals, src2d, dst2d)


def _tc1_body(deg0_ref, deg1_ref, x_ref, dis_ref, s1_ref):
    deg = deg0_ref[...] + deg1_ref[...] + 1.0
    dis = jnp.where(deg > 0.0, lax.rsqrt(deg), 0.0)
    dis_ref[...] = dis
    s1_ref[...] = dis * x_ref[...]


def _tc2_body(hid_dim, a0_ref, a1_ref, dis_ref, x_ref, w1_ref, b1_ref,
              w2_ref, s2_ref, v_ref):
    dis = dis_ref[...]
    p1 = dis * (a0_ref[...] + a1_ref[...]) + dis * dis * x_ref[...]
    v = jnp.zeros_like(p1)
    for j in range(hid_dim):
        v = v + jnp.maximum(p1 * w1_ref[0, j] + b1_ref[j], 0.0) * w2_ref[j, 0]
    v_ref[...] = v
    s2_ref[...] = dis * v


def _tc3_body(a0_ref, a1_ref, dis_ref, v_ref, b2_ref, out_ref):
    dis = dis_ref[...]
    out_ref[...] = (dis * (a0_ref[...] + a1_ref[...])
                    + dis * dis * v_ref[...] + b2_ref[0])


_NODE2D = jax.ShapeDtypeStruct((ROWS2D, 128), _f32)


def _smem_spec():
    return pl.BlockSpec(memory_space=pltpu.SMEM)


@jax.jit
def _tc1(deg0, deg1, x2d):
    return pl.pallas_call(
        _tc1_body,
        out_shape=(_NODE2D, _NODE2D),
    )(deg0, deg1, x2d)


@jax.jit
def _tc2(a0, a1, dis, x2d, W1, b1, W2):
    return pl.pallas_call(
        functools.partial(_tc2_body, W1.shape[1]),
        in_specs=[pl.BlockSpec(memory_space=pltpu.VMEM)] * 4
        + [_smem_spec(), _smem_spec(), _smem_spec()],
        out_shape=(_NODE2D, _NODE2D),
    )(a0, a1, dis, x2d, W1, b1, W2)


@jax.jit
def _tc3(a0, a1, dis, v, b2):
    return pl.pallas_call(
        _tc3_body,
        in_specs=[pl.BlockSpec(memory_space=pltpu.VMEM)] * 4 + [_smem_spec()],
        out_shape=_NODE2D,
    )(a0, a1, dis, v, b2)


def kernel(x, edge_index, W1, b1, W2, b2):
    n = x.shape[0]
    e = edge_index.shape[1]
    nchunks = e // (CHUNK * NW)
    src2d = edge_index[0].astype(jnp.int32)
    dst2d = edge_index[1].astype(jnp.int32)
    x2d = jnp.pad(x[:, 0], (0, N_PAD - n)).reshape(ROWS2D, 128)

    deg_p = _deg_pass(dst2d, nchunks)
    d2 = deg_p.reshape(NC, ROWS2D, 128)
    dis, s1 = _tc1(d2[0], d2[1], x2d)

    a1_p = _edge_pass(s1.reshape(N_PAD), src2d, dst2d, nchunks)
    a1 = a1_p.reshape(NC, ROWS2D, 128)
    s2, v = _tc2(a1[0], a1[1], dis, x2d, W1, b1, W2)

    a2_p = _edge_pass(s2.reshape(N_PAD), src2d, dst2d, nchunks)
    a2 = a2_p.reshape(NC, ROWS2D, 128)
    out2d = _tc3(a2[0], a2[1], dis, v, b2)

    return out2d.reshape(N_PAD)[:n, None]


# speedup vs baseline: 469.3024x; 1.9732x over previous
import functools

import jax
import jax.numpy as jnp
from jax import lax
from jax.experimental import pallas as pl
from jax.experimental.pallas import tpu as pltpu
from jax.experimental.pallas import tpu_sc as plsc

N_NODES = 100000
N_PAD = 100352
ROWS2D = 784
WSEG = N_PAD // 16

LANES = 16
NC = 2
NS = 16
NW = NC * NS
CH_ROWS = 125
CHUNK = CH_ROWS * LANES
NBUF = 3

_f32 = jnp.float32


def _mesh():
    return plsc.VectorSubcoreMesh(core_axis_name="c", subcore_axis_name="s")


def _zero_fill(wbuf):
    z = jnp.zeros((LANES,), _f32)
    for k in range(WSEG // LANES):
        wbuf[pl.ds(k * LANES, LANES)] = z


def _acc_init(s, wbuf, acc):
    _zero_fill(wbuf)
    pltpu.sync_copy(wbuf, acc.at[pl.ds(s * WSEG, WSEG)])


def _acc_writeout(c, s, wbuf, acc, out_hbm):
    pltpu.sync_copy(acc.at[pl.ds(s * WSEG, WSEG)], wbuf)
    pltpu.sync_copy(wbuf, out_hbm.at[c, pl.ds(s * WSEG, WSEG)])


def _deg_body(nchunks, dst_hbm, out_hbm, dbuf0, dbuf1, dbuf2, dbuf3,
              vbuf, wbuf, acc, sin, sout):
    dbuf = [dbuf0, dbuf1, dbuf2, dbuf3]
    c = lax.axis_index("c")
    s = lax.axis_index("s")
    one = jnp.ones((LANES,), _f32)
    for k in range(CH_ROWS):
        vbuf[pl.ds(k * LANES, LANES)] = one
    _acc_init(s, wbuf, acc)
    plsc.subcore_barrier()
    base0 = (c * NS + s) * (nchunks * CHUNK)

    def start_in(j, b4):
        pltpu.async_copy(dst_hbm.at[pl.ds(base0 + j * CHUNK, CHUNK)],
                         dbuf[b4], sin[b4])

    def wait_in(b4):
        pltpu.make_async_copy(dst_hbm.at[pl.ds(0, CHUNK)], dbuf[b4],
                              sin[b4]).wait()

    def wait_out(b2):
        pltpu.make_async_copy(vbuf, acc.at[dbuf[0]], sout[b2]).wait()

    def visit(j, b4, b2, do_wait_out, prefetch_guard):
        wait_in(b4)
        if do_wait_out:
            wait_out(b2)
        pltpu.async_copy(vbuf, acc.at[dbuf[b4]], sout[b2], add=True)
        if prefetch_guard is not None:
            @pl.when(prefetch_guard)
            def _():
                start_in(j + 2, (b4 + 2) % 4)

    start_in(0, 0)
    start_in(1, 1)
    visit(jnp.int32(0), 0, 0, False, True)
    visit(jnp.int32(1), 1, 1, False, True)

    def loop(i2, carry):
        for t in range(4):
            j = 4 * i2 + 2 + t
            visit(j, (2 + t) % 4, t % 2, True, j + 2 < nchunks)
        return carry

    lax.fori_loop(0, (nchunks - 2) // 4, loop, 0)
    wait_out(0)
    wait_out(1)
    plsc.subcore_barrier()
    _acc_writeout(c, s, wbuf, acc, out_hbm)


def _edge_body(nchunks, vals_hbm, src_hbm, dst_hbm, out_hbm,
               sbuf0, sbuf1, dbuf0, dbuf1, dbuf2, dbuf3, vbuf0, vbuf1,
               wbuf, s_local, acc, sin_s, sin_d, sout):
    sbuf = [sbuf0, sbuf1]
    dbuf = [dbuf0, dbuf1, dbuf2, dbuf3]
    vbuf = [vbuf0, vbuf1]
    c = lax.axis_index("c")
    s = lax.axis_index("s")
    pltpu.sync_copy(vals_hbm, s_local)
    _acc_init(s, wbuf, acc)
    plsc.subcore_barrier()
    base0 = (c * NS + s) * (nchunks * CHUNK)

    def start_in(j, b4, b2):
        r = base0 + j * CHUNK
        pltpu.async_copy(src_hbm.at[pl.ds(r, CHUNK)], sbuf[b2], sin_s[b2])
        pltpu.async_copy(dst_hbm.at[pl.ds(r, CHUNK)], dbuf[b4],
                         sin_d[b4])

    def wait_in(b4, b2):
        pltpu.make_async_copy(src_hbm.at[pl.ds(0, CHUNK)], sbuf[b2],
                              sin_s[b2]).wait()
        pltpu.make_async_copy(dst_hbm.at[pl.ds(0, CHUNK)], dbuf[b4],
                              sin_d[b4]).wait()

    def wait_out(b2):
        pltpu.make_async_copy(vbuf[b2], acc.at[dbuf[0]], sout[b2]).wait()

    def visit(j, b4, b2, do_wait_out, prefetch_guard):
        wait_in(b4, b2)
        if do_wait_out:
            wait_out(b2)
        for k in range(CH_ROWS):
            idx = sbuf[b2][pl.ds(k * LANES, LANES)]
            vbuf[b2][pl.ds(k * LANES, LANES)] = plsc.load_gather(
                s_local, [idx])
        pltpu.async_copy(vbuf[b2], acc.at[dbuf[b4]], sout[b2], add=True)
        if prefetch_guard is not None:
            @pl.when(prefetch_guard)
            def _():
                start_in(j + 2, (b4 + 2) % 4, b2)
        return

    start_in(0, 0, 0)
    start_in(1, 1, 1)
    visit(jnp.int32(0), 0, 0, False, True)
    visit(jnp.int32(1), 1, 1, False, True)

    def loop(i2, carry):
        for t in range(4):
            j = 4 * i2 + 2 + t
            visit(j, (2 + t) % 4, t % 2, True, j + 2 < nchunks)
        return carry

    lax.fori_loop(0, (nchunks - 2) // 4, loop, 0)
    wait_out(0)
    wait_out(1)
    plsc.subcore_barrier()
    _acc_writeout(c, s, wbuf, acc, out_hbm)


@functools.partial(jax.jit, static_argnames=("nchunks",))
def _deg_pass(dst, nchunks):
    return pl.kernel(
        functools.partial(_deg_body, nchunks),
        out_type=jax.ShapeDtypeStruct((NC, N_PAD), _f32),
        mesh=_mesh(),
        compiler_params=pltpu.CompilerParams(needs_layout_passes=False),
        scratch_types=[
            pltpu.VMEM((CHUNK,), jnp.int32),
            pltpu.VMEM((CHUNK,), jnp.int32),
            pltpu.VMEM((CHUNK,), jnp.int32),
            pltpu.VMEM((CHUNK,), jnp.int32),
            pltpu.VMEM((CHUNK,), _f32),
            pltpu.VMEM((WSEG,), _f32),
            pltpu.VMEM_SHARED((N_PAD,), _f32),
            [pltpu.SemaphoreType.DMA] * 4,
            [pltpu.SemaphoreType.DMA] * 2,
        ],
    )(dst)


@functools.partial(jax.jit, static_argnames=("nchunks",))
def _edge_pass(vals, src, dst, nchunks):
    return pl.kernel(
        functools.partial(_edge_body, nchunks),
        out_type=jax.ShapeDtypeStruct((NC, N_PAD), _f32),
        mesh=_mesh(),
        compiler_params=pltpu.CompilerParams(needs_layout_passes=False),
        scratch_types=[
            pltpu.VMEM((CHUNK,), jnp.int32),
            pltpu.VMEM((CHUNK,), jnp.int32),
            pltpu.VMEM((CHUNK,), jnp.int32),
            pltpu.VMEM((CHUNK,), jnp.int32),
            pltpu.VMEM((CHUNK,), jnp.int32),
            pltpu.VMEM((CHUNK,), jnp.int32),
            pltpu.VMEM((CHUNK,), _f32),
            pltpu.VMEM((CHUNK,), _f32),
            pltpu.VMEM((WSEG,), _f32),
            pltpu.VMEM((N_PAD,), _f32),
            pltpu.VMEM_SHARED((N_PAD,), _f32),
            [pltpu.SemaphoreType.DMA] * 2,
            [pltpu.SemaphoreType.DMA] * 4,
            [pltpu.SemaphoreType.DMA] * 2,
        ],
    )(vals, src, dst)


def _tc1_body(degp_ref, x_ref, dis_ref, s1_ref):
    deg = degp_ref[0] + degp_ref[1] + 1.0
    dis = jnp.where(deg > 0.0, lax.rsqrt(deg), 0.0)
    dis_ref[...] = dis
    s1_ref[...] = dis * x_ref[...]


def _tc2_body(hid_dim, ap_ref, dis_ref, x_ref, w1_ref, b1_ref,
              w2_ref, s2_ref, v_ref):
    dis = dis_ref[...]
    p1 = dis * (ap_ref[0] + ap_ref[1]) + dis * dis * x_ref[...]
    v = jnp.zeros_like(p1)
    for j in range(hid_dim):
        v = v + jnp.maximum(p1 * w1_ref[0, j] + b1_ref[j], 0.0) * w2_ref[j, 0]
    v_ref[...] = v
    s2_ref[...] = dis * v


def _tc3_body(ap_ref, dis_ref, v_ref, b2_ref, out_ref):
    dis = dis_ref[...]
    out_ref[...] = (dis * (ap_ref[0] + ap_ref[1])
                    + dis * dis * v_ref[...] + b2_ref[0])


_NODE2D = jax.ShapeDtypeStruct((ROWS2D, 128), _f32)


def _smem_spec():
    return pl.BlockSpec(memory_space=pltpu.SMEM)


@jax.jit
def _tc1(degp, x2d):
    return pl.pallas_call(
        _tc1_body,
        out_shape=(_NODE2D, _NODE2D),
    )(degp, x2d)


@jax.jit
def _tc2(ap, dis, x2d, W1, b1, W2):
    return pl.pallas_call(
        functools.partial(_tc2_body, W1.shape[1]),
        in_specs=[pl.BlockSpec(memory_space=pltpu.VMEM)] * 3
        + [_smem_spec(), _smem_spec(), _smem_spec()],
        out_shape=(_NODE2D, _NODE2D),
    )(ap, dis, x2d, W1, b1, W2)


@jax.jit
def _tc3(ap, dis, v, b2):
    return pl.pallas_call(
        _tc3_body,
        in_specs=[pl.BlockSpec(memory_space=pltpu.VMEM)] * 3 + [_smem_spec()],
        out_shape=_NODE2D,
    )(ap, dis, v, b2)


def kernel(x, edge_index, W1, b1, W2, b2):
    n = x.shape[0]
    e = edge_index.shape[1]
    nchunks = e // (CHUNK * NW)
    ei = edge_index.astype(jnp.int32)
    src, dst = ei[0], ei[1]
    x2d = jnp.pad(x[:, 0], (0, N_PAD - n)).reshape(ROWS2D, 128)

    deg_p = _deg_pass(dst, nchunks)
    dis, s1 = _tc1(deg_p.reshape(NC, ROWS2D, 128), x2d)

    a1_p = _edge_pass(s1.reshape(N_PAD), src, dst, nchunks)
    s2, v = _tc2(a1_p.reshape(NC, ROWS2D, 128), dis, x2d, W1, b1, W2)

    a2_p = _edge_pass(s2.reshape(N_PAD), src, dst, nchunks)
    out2d = _tc3(a2_p.reshape(NC, ROWS2D, 128), dis, v, b2)

    return out2d.reshape(N_PAD)[:n, None]


# speedup vs baseline: 515.7885x; 1.0991x over previous
import functools

import jax
import jax.numpy as jnp
from jax import lax
from jax.experimental import pallas as pl
from jax.experimental.pallas import tpu as pltpu
from jax.experimental.pallas import tpu_sc as plsc

N_NODES = 100000
N_PAD = 100352
ROWS2D = 784
WSEG = N_PAD // 16

LANES = 16
NC = 2
NS = 16
NW = NC * NS
CH_ROWS = 125
CHUNK = CH_ROWS * LANES
NBUF = 3

_f32 = jnp.float32


def _mesh():
    return plsc.VectorSubcoreMesh(core_axis_name="c", subcore_axis_name="s")


def _zero_fill(wbuf):
    z = jnp.zeros((LANES,), _f32)
    for k in range(WSEG // LANES):
        wbuf[pl.ds(k * LANES, LANES)] = z


def _acc_init(s, wbuf, acc):
    _zero_fill(wbuf)
    pltpu.sync_copy(wbuf, acc.at[pl.ds(s * WSEG, WSEG)])


def _acc_writeout(c, s, wbuf, acc, out_hbm):
    pltpu.sync_copy(acc.at[pl.ds(s * WSEG, WSEG)], wbuf)
    pltpu.sync_copy(wbuf, out_hbm.at[c, pl.ds(s * WSEG, WSEG)])


def _deg_body(nchunks, dst_hbm, out_hbm, dbuf0, dbuf1, dbuf2, dbuf3,
              vbuf, wbuf, acc, sin, sout):
    dbuf = [dbuf0, dbuf1, dbuf2, dbuf3]
    c = lax.axis_index("c")
    s = lax.axis_index("s")
    one = jnp.ones((LANES,), _f32)
    for k in range(CH_ROWS):
        vbuf[pl.ds(k * LANES, LANES)] = one
    _acc_init(s, wbuf, acc)
    plsc.subcore_barrier()
    base0 = (c * NS + s) * (nchunks * CHUNK)

    def start_in(j, b4):
        pltpu.async_copy(dst_hbm.at[pl.ds(base0 + j * CHUNK, CHUNK)],
                         dbuf[b4], sin[b4])

    def wait_in(b4):
        pltpu.make_async_copy(dst_hbm.at[pl.ds(0, CHUNK)], dbuf[b4],
                              sin[b4]).wait()

    def wait_out(b2):
        pltpu.make_async_copy(vbuf, acc.at[dbuf[0]], sout[b2]).wait()

    def visit(j, b4, b2, do_wait_out, prefetch_guard):
        wait_in(b4)
        if do_wait_out:
            wait_out(b2)
        pltpu.async_copy(vbuf, acc.at[dbuf[b4]], sout[b2], add=True)
        if prefetch_guard is not None:
            @pl.when(prefetch_guard)
            def _():
                start_in(j + 2, (b4 + 2) % 4)

    start_in(0, 0)
    start_in(1, 1)
    visit(jnp.int32(0), 0, 0, False, True)
    visit(jnp.int32(1), 1, 1, False, True)

    def loop(i2, carry):
        for t in range(4):
            j = 4 * i2 + 2 + t
            visit(j, (2 + t) % 4, t % 2, True, j + 2 < nchunks)
        return carry

    lax.fori_loop(0, (nchunks - 2) // 4, loop, 0)
    wait_out(0)
    wait_out(1)
    plsc.subcore_barrier()
    _acc_writeout(c, s, wbuf, acc, out_hbm)


def _edge_body(nchunks, vals_hbm, src_hbm, dst_hbm, out_hbm,
               sbuf0, sbuf1, dbuf0, dbuf1, dbuf2, dbuf3, vbuf0, vbuf1,
               wbuf, s_local, acc, sin_s, sin_d, sout):
    sbuf = [sbuf0, sbuf1]
    dbuf = [dbuf0, dbuf1, dbuf2, dbuf3]
    vbuf = [vbuf0, vbuf1]
    c = lax.axis_index("c")
    s = lax.axis_index("s")
    pltpu.sync_copy(vals_hbm, s_local)
    _acc_init(s, wbuf, acc)
    plsc.subcore_barrier()
    base0 = (c * NS + s) * (nchunks * CHUNK)

    def start_in(j, b4, b2):
        r = base0 + j * CHUNK
        pltpu.async_copy(src_hbm.at[pl.ds(r, CHUNK)], sbuf[b2], sin_s[b2])
        pltpu.async_copy(dst_hbm.at[pl.ds(r, CHUNK)], dbuf[b4],
                         sin_d[b4])

    def wait_in(b4, b2):
        pltpu.make_async_copy(src_hbm.at[pl.ds(0, CHUNK)], sbuf[b2],
                              sin_s[b2]).wait()
        pltpu.make_async_copy(dst_hbm.at[pl.ds(0, CHUNK)], dbuf[b4],
                              sin_d[b4]).wait()

    def wait_out(b2):
        pltpu.make_async_copy(vbuf[b2], acc.at[dbuf[0]], sout[b2]).wait()

    def visit(j, b4, b2, do_wait_out, prefetch_guard):
        wait_in(b4, b2)
        if do_wait_out:
            wait_out(b2)
        for k0 in range(0, CH_ROWS, 5):
            idxs = [sbuf[b2][pl.ds((k0 + i) * LANES, LANES)]
                    for i in range(5)]
            vals = [plsc.load_gather(s_local, [ix]) for ix in idxs]
            for i in range(5):
                vbuf[b2][pl.ds((k0 + i) * LANES, LANES)] = vals[i]
        pltpu.async_copy(vbuf[b2], acc.at[dbuf[b4]], sout[b2], add=True)
        if prefetch_guard is not None:
            @pl.when(prefetch_guard)
            def _():
                start_in(j + 2, (b4 + 2) % 4, b2)
        return

    start_in(0, 0, 0)
    start_in(1, 1, 1)
    visit(jnp.int32(0), 0, 0, False, True)
    visit(jnp.int32(1), 1, 1, False, True)

    def loop(i2, carry):
        for t in range(4):
            j = 4 * i2 + 2 + t
            visit(j, (2 + t) % 4, t % 2, True, j + 2 < nchunks)
        return carry

    lax.fori_loop(0, (nchunks - 2) // 4, loop, 0)
    wait_out(0)
    wait_out(1)
    plsc.subcore_barrier()
    _acc_writeout(c, s, wbuf, acc, out_hbm)


@functools.partial(jax.jit, static_argnames=("nchunks",))
def _deg_pass(dst, nchunks):
    return pl.kernel(
        functools.partial(_deg_body, nchunks),
        out_type=jax.ShapeDtypeStruct((NC, N_PAD), _f32),
        mesh=_mesh(),
        compiler_params=pltpu.CompilerParams(needs_layout_passes=False),
        scratch_types=[
            pltpu.VMEM((CHUNK,), jnp.int32),
            pltpu.VMEM((CHUNK,), jnp.int32),
            pltpu.VMEM((CHUNK,), jnp.int32),
            pltpu.VMEM((CHUNK,), jnp.int32),
            pltpu.VMEM((CHUNK,), _f32),
            pltpu.VMEM((WSEG,), _f32),
            pltpu.VMEM_SHARED((N_PAD,), _f32),
            [pltpu.SemaphoreType.DMA] * 4,
            [pltpu.SemaphoreType.DMA] * 2,
        ],
    )(dst)


@functools.partial(jax.jit, static_argnames=("nchunks",))
def _edge_pass(vals, src, dst, nchunks):
    return pl.kernel(
        functools.partial(_edge_body, nchunks),
        out_type=jax.ShapeDtypeStruct((NC, N_PAD), _f32),
        mesh=_mesh(),
        compiler_params=pltpu.CompilerParams(needs_layout_passes=False),
        scratch_types=[
            pltpu.VMEM((CHUNK,), jnp.int32),
            pltpu.VMEM((CHUNK,), jnp.int32),
            pltpu.VMEM((CHUNK,), jnp.int32),
            pltpu.VMEM((CHUNK,), jnp.int32),
            pltpu.VMEM((CHUNK,), jnp.int32),
            pltpu.VMEM((CHUNK,), jnp.int32),
            pltpu.VMEM((CHUNK,), _f32),
            pltpu.VMEM((CHUNK,), _f32),
            pltpu.VMEM((WSEG,), _f32),
            pltpu.VMEM((N_PAD,), _f32),
            pltpu.VMEM_SHARED((N_PAD,), _f32),
            [pltpu.SemaphoreType.DMA] * 2,
            [pltpu.SemaphoreType.DMA] * 4,
            [pltpu.SemaphoreType.DMA] * 2,
        ],
    )(vals, src, dst)


def _tc1_body(degp_ref, x_ref, dis_ref, s1_ref):
    deg = degp_ref[0] + degp_ref[1] + 1.0
    dis = jnp.where(deg > 0.0, lax.rsqrt(deg), 0.0)
    dis_ref[...] = dis
    s1_ref[...] = dis * x_ref[...]


def _tc2_body(hid_dim, ap_ref, dis_ref, x_ref, w1_ref, b1_ref,
              w2_ref, s2_ref, v_ref):
    dis = dis_ref[...]
    p1 = dis * (ap_ref[0] + ap_ref[1]) + dis * dis * x_ref[...]
    v = jnp.zeros_like(p1)
    for j in range(hid_dim):
        v = v + jnp.maximum(p1 * w1_ref[0, j] + b1_ref[j], 0.0) * w2_ref[j, 0]
    v_ref[...] = v
    s2_ref[...] = dis * v


def _tc3_body(ap_ref, dis_ref, v_ref, b2_ref, out_ref):
    dis = dis_ref[...]
    out_ref[...] = (dis * (ap_ref[0] + ap_ref[1])
                    + dis * dis * v_ref[...] + b2_ref[0])


_NODE2D = jax.ShapeDtypeStruct((ROWS2D, 128), _f32)


def _smem_spec():
    return pl.BlockSpec(memory_space=pltpu.SMEM)


@jax.jit
def _tc1(degp, x2d):
    return pl.pallas_call(
        _tc1_body,
        out_shape=(_NODE2D, _NODE2D),
    )(degp, x2d)


@jax.jit
def _tc2(ap, dis, x2d, W1, b1, W2):
    return pl.pallas_call(
        functools.partial(_tc2_body, W1.shape[1]),
        in_specs=[pl.BlockSpec(memory_space=pltpu.VMEM)] * 3
        + [_smem_spec(), _smem_spec(), _smem_spec()],
        out_shape=(_NODE2D, _NODE2D),
    )(ap, dis, x2d, W1, b1, W2)


@jax.jit
def _tc3(ap, dis, v, b2):
    return pl.pallas_call(
        _tc3_body,
        in_specs=[pl.BlockSpec(memory_space=pltpu.VMEM)] * 3 + [_smem_spec()],
        out_shape=_NODE2D,
    )(ap, dis, v, b2)


def kernel(x, edge_index, W1, b1, W2, b2):
    n = x.shape[0]
    e = edge_index.shape[1]
    nchunks = e // (CHUNK * NW)
    ei = edge_index.astype(jnp.int32)
    src, dst = ei[0], ei[1]
    x2d = jnp.pad(x[:, 0], (0, N_PAD - n)).reshape(ROWS2D, 128)

    deg_p = _deg_pass(dst, nchunks)
    dis, s1 = _tc1(deg_p.reshape(NC, ROWS2D, 128), x2d)

    a1_p = _edge_pass(s1.reshape(N_PAD), src, dst, nchunks)
    s2, v = _tc2(a1_p.reshape(NC, ROWS2D, 128), dis, x2d, W1, b1, W2)

    a2_p = _edge_pass(s2.reshape(N_PAD), src, dst, nchunks)
    out2d = _tc3(a2_p.reshape(NC, ROWS2D, 128), dis, v, b2)

    return out2d.reshape(N_PAD)[:n, None]


# speedup vs baseline: 517.0180x; 1.0024x over previous
import functools

import jax
import jax.numpy as jnp
from jax import lax
from jax.experimental import pallas as pl
from jax.experimental.pallas import tpu as pltpu
from jax.experimental.pallas import tpu_sc as plsc

N_NODES = 100000
N_PAD = 100352
ROWS2D = 784
WSEG = N_PAD // 16

LANES = 16
NC = 2
NS = 16
NW = NC * NS
CH_ROWS = 125
CHUNK = CH_ROWS * LANES
NBUF = 3

_f32 = jnp.float32


def _mesh():
    return plsc.VectorSubcoreMesh(core_axis_name="c", subcore_axis_name="s")


def _zero_fill(wbuf):
    z = jnp.zeros((LANES,), _f32)
    for k in range(WSEG // LANES):
        wbuf[pl.ds(k * LANES, LANES)] = z


def _acc_init(s, wbuf, acc):
    _zero_fill(wbuf)
    pltpu.sync_copy(wbuf, acc.at[pl.ds(s * WSEG, WSEG)])


def _acc_writeout(c, s, wbuf, acc, out_hbm):
    pltpu.sync_copy(acc.at[pl.ds(s * WSEG, WSEG)], wbuf)
    pltpu.sync_copy(wbuf, out_hbm.at[c, pl.ds(s * WSEG, WSEG)])


def _deg_body(nchunks, dst_hbm, out_hbm, dbuf0, dbuf1, dbuf2, dbuf3,
              vbuf, wbuf, acc, sin, sout):
    dbuf = [dbuf0, dbuf1, dbuf2, dbuf3]
    c = lax.axis_index("c")
    s = lax.axis_index("s")
    one = jnp.ones((LANES,), _f32)
    for k in range(CH_ROWS):
        vbuf[pl.ds(k * LANES, LANES)] = one
    _acc_init(s, wbuf, acc)
    plsc.subcore_barrier()
    base0 = (c * NS + s) * (nchunks * CHUNK)

    def start_in(j, b4):
        pltpu.async_copy(dst_hbm.at[pl.ds(base0 + j * CHUNK, CHUNK)],
                         dbuf[b4], sin[b4])

    def wait_in(b4):
        pltpu.make_async_copy(dst_hbm.at[pl.ds(0, CHUNK)], dbuf[b4],
                              sin[b4]).wait()

    def wait_out(b2):
        pltpu.make_async_copy(vbuf, acc.at[dbuf[0]], sout[b2]).wait()

    def visit(j, b4, b2, do_wait_out, prefetch_guard):
        wait_in(b4)
        if do_wait_out:
            wait_out(b2)
        pltpu.async_copy(vbuf, acc.at[dbuf[b4]], sout[b2], add=True)
        if prefetch_guard is not None:
            @pl.when(prefetch_guard)
            def _():
                start_in(j + 2, (b4 + 2) % 4)

    start_in(0, 0)
    start_in(1, 1)
    visit(jnp.int32(0), 0, 0, False, True)
    visit(jnp.int32(1), 1, 1, False, True)

    def loop(i2, carry):
        for t in range(4):
            j = 4 * i2 + 2 + t
            visit(j, (2 + t) % 4, t % 2, True, j + 2 < nchunks)
        return carry

    lax.fori_loop(0, (nchunks - 2) // 4, loop, 0)
    wait_out(0)
    wait_out(1)
    plsc.subcore_barrier()
    _acc_writeout(c, s, wbuf, acc, out_hbm)


def _edge_body(nchunks, vals_hbm, src_hbm, dst_hbm, out_hbm,
               sbuf0, sbuf1, dbuf0, dbuf1, dbuf2, dbuf3, vbuf0, vbuf1,
               wbuf, s_local, acc, sin_s, sin_d, sout):
    sbuf = [sbuf0, sbuf1]
    dbuf = [dbuf0, dbuf1, dbuf2, dbuf3]
    vbuf = [vbuf0, vbuf1]
    c = lax.axis_index("c")
    s = lax.axis_index("s")
    pltpu.sync_copy(vals_hbm, s_local)
    _acc_init(s, wbuf, acc)
    plsc.subcore_barrier()
    base0 = (c * NS + s) * (nchunks * CHUNK)

    def start_in(j, b4, b2):
        r = base0 + j * CHUNK
        pltpu.async_copy(src_hbm.at[pl.ds(r, CHUNK)], sbuf[b2], sin_s[b2])
        pltpu.async_copy(dst_hbm.at[pl.ds(r, CHUNK)], dbuf[b4],
                         sin_d[b4])

    def wait_in(b4, b2):
        pltpu.make_async_copy(src_hbm.at[pl.ds(0, CHUNK)], sbuf[b2],
                              sin_s[b2]).wait()
        pltpu.make_async_copy(dst_hbm.at[pl.ds(0, CHUNK)], dbuf[b4],
                              sin_d[b4]).wait()

    def wait_out(b2):
        pltpu.make_async_copy(vbuf[b2], acc.at[dbuf[0]], sout[b2]).wait()

    def visit(j, b4, b2, do_wait_out, prefetch_guard):
        wait_in(b4, b2)
        if do_wait_out:
            wait_out(b2)
        for k0 in range(0, CH_ROWS, 25):
            idxs = [sbuf[b2][pl.ds((k0 + i) * LANES, LANES)]
                    for i in range(25)]
            vals = [plsc.load_gather(s_local, [ix]) for ix in idxs]
            for i in range(25):
                vbuf[b2][pl.ds((k0 + i) * LANES, LANES)] = vals[i]
        pltpu.async_copy(vbuf[b2], acc.at[dbuf[b4]], sout[b2], add=True)
        if prefetch_guard is not None:
            @pl.when(prefetch_guard)
            def _():
                start_in(j + 2, (b4 + 2) % 4, b2)
        return

    start_in(0, 0, 0)
    start_in(1, 1, 1)
    visit(jnp.int32(0), 0, 0, False, True)
    visit(jnp.int32(1), 1, 1, False, True)

    def loop(i2, carry):
        for t in range(4):
            j = 4 * i2 + 2 + t
            visit(j, (2 + t) % 4, t % 2, True, j + 2 < nchunks)
        return carry

    lax.fori_loop(0, (nchunks - 2) // 4, loop, 0)
    wait_out(0)
    wait_out(1)
    plsc.subcore_barrier()
    _acc_writeout(c, s, wbuf, acc, out_hbm)


@functools.partial(jax.jit, static_argnames=("nchunks",))
def _deg_pass(dst, nchunks):
    return pl.kernel(
        functools.partial(_deg_body, nchunks),
        out_type=jax.ShapeDtypeStruct((NC, N_PAD), _f32),
        mesh=_mesh(),
        compiler_params=pltpu.CompilerParams(needs_layout_passes=False),
        scratch_types=[
            pltpu.VMEM((CHUNK,), jnp.int32),
            pltpu.VMEM((CHUNK,), jnp.int32),
            pltpu.VMEM((CHUNK,), jnp.int32),
            pltpu.VMEM((CHUNK,), jnp.int32),
            pltpu.VMEM((CHUNK,), _f32),
            pltpu.VMEM((WSEG,), _f32),
            pltpu.VMEM_SHARED((N_PAD,), _f32),
            [pltpu.SemaphoreType.DMA] * 4,
            [pltpu.SemaphoreType.DMA] * 2,
        ],
    )(dst)


@functools.partial(jax.jit, static_argnames=("nchunks",))
def _edge_pass(vals, src, dst, nchunks):
    return pl.kernel(
        functools.partial(_edge_body, nchunks),
        out_type=jax.ShapeDtypeStruct((NC, N_PAD), _f32),
        mesh=_mesh(),
        compiler_params=pltpu.CompilerParams(needs_layout_passes=False),
        scratch_types=[
            pltpu.VMEM((CHUNK,), jnp.int32),
            pltpu.VMEM((CHUNK,), jnp.int32),
            pltpu.VMEM((CHUNK,), jnp.int32),
            pltpu.VMEM((CHUNK,), jnp.int32),
            pltpu.VMEM((CHUNK,), jnp.int32),
            pltpu.VMEM((CHUNK,), jnp.int32),
            pltpu.VMEM((CHUNK,), _f32),
            pltpu.VMEM((CHUNK,), _f32),
            pltpu.VMEM((WSEG,), _f32),
            pltpu.VMEM((N_PAD,), _f32),
            pltpu.VMEM_SHARED((N_PAD,), _f32),
            [pltpu.SemaphoreType.DMA] * 2,
            [pltpu.SemaphoreType.DMA] * 4,
            [pltpu.SemaphoreType.DMA] * 2,
        ],
    )(vals, src, dst)


def _tc1_body(degp_ref, x_ref, dis_ref, s1_ref):
    deg = degp_ref[0] + degp_ref[1] + 1.0
    dis = jnp.where(deg > 0.0, lax.rsqrt(deg), 0.0)
    dis_ref[...] = dis
    s1_ref[...] = dis * x_ref[...]


def _tc2_body(hid_dim, ap_ref, dis_ref, x_ref, w1_ref, b1_ref,
              w2_ref, s2_ref, v_ref):
    dis = dis_ref[...]
    p1 = dis * (ap_ref[0] + ap_ref[1]) + dis * dis * x_ref[...]
    v = jnp.zeros_like(p1)
    for j in range(hid_dim):
        v = v + jnp.maximum(p1 * w1_ref[0, j] + b1_ref[j], 0.0) * w2_ref[j, 0]
    v_ref[...] = v
    s2_ref[...] = dis * v


def _tc3_body(ap_ref, dis_ref, v_ref, b2_ref, out_ref):
    dis = dis_ref[...]
    out_ref[...] = (dis * (ap_ref[0] + ap_ref[1])
                    + dis * dis * v_ref[...] + b2_ref[0])


_NODE2D = jax.ShapeDtypeStruct((ROWS2D, 128), _f32)


def _smem_spec():
    return pl.BlockSpec(memory_space=pltpu.SMEM)


@jax.jit
def _tc1(degp, x2d):
    return pl.pallas_call(
        _tc1_body,
        out_shape=(_NODE2D, _NODE2D),
    )(degp, x2d)


@jax.jit
def _tc2(ap, dis, x2d, W1, b1, W2):
    return pl.pallas_call(
        functools.partial(_tc2_body, W1.shape[1]),
        in_specs=[pl.BlockSpec(memory_space=pltpu.VMEM)] * 3
        + [_smem_spec(), _smem_spec(), _smem_spec()],
        out_shape=(_NODE2D, _NODE2D),
    )(ap, dis, x2d, W1, b1, W2)


@jax.jit
def _tc3(ap, dis, v, b2):
    return pl.pallas_call(
        _tc3_body,
        in_specs=[pl.BlockSpec(memory_space=pltpu.VMEM)] * 3 + [_smem_spec()],
        out_shape=_NODE2D,
    )(ap, dis, v, b2)


def kernel(x, edge_index, W1, b1, W2, b2):
    n = x.shape[0]
    e = edge_index.shape[1]
    nchunks = e // (CHUNK * NW)
    ei = edge_index.astype(jnp.int32)
    src, dst = ei[0], ei[1]
    x2d = jnp.pad(x[:, 0], (0, N_PAD - n)).reshape(ROWS2D, 128)

    deg_p = _deg_pass(dst, nchunks)
    dis, s1 = _tc1(deg_p.reshape(NC, ROWS2D, 128), x2d)

    a1_p = _edge_pass(s1.reshape(N_PAD), src, dst, nchunks)
    s2, v = _tc2(a1_p.reshape(NC, ROWS2D, 128), dis, x2d, W1, b1, W2)

    a2_p = _edge_pass(s2.reshape(N_PAD), src, dst, nchunks)
    out2d = _tc3(a2_p.reshape(NC, ROWS2D, 128), dis, v, b2)

    return out2d.reshape(N_PAD)[:n, None]
